# baseline (speedup 1.0000x reference)
import functools

import jax
import jax.numpy as jnp
from jax import lax
from jax.experimental import pallas as pl
from jax.experimental.pallas import tpu as pltpu
from jax.experimental.pallas import tpu_sc as plsc

N_FIELDS = 26
CARD = 100000
DIM = 24
BATCH = 16384
OUT_W = N_FIELDS * DIM

NC = 2
NS = 16
NW = NC * NS
IDX_W = BATCH * N_FIELDS // NW
PERIOD = 208
BLK_IDX = 416
NBLK = IDX_W // BLK_IDX

assert IDX_W % BLK_IDX == 0 and BLK_IDX % PERIOD == 0

_mesh = plsc.VectorSubcoreMesh(core_axis_name="c", subcore_axis_name="s")


@functools.partial(
    pl.kernel,
    mesh=_mesh,
    out_type=jax.ShapeDtypeStruct((BATCH * N_FIELDS, DIM), jnp.float32),
    scratch_types=[
        pltpu.VMEM((IDX_W,), jnp.int32),
        pltpu.VMEM((2, BLK_IDX, DIM), jnp.float32),
        pltpu.SemaphoreType.DMA,
        pltpu.SemaphoreType.DMA,
    ],
)
def _embed_gather(x_ref, table_ref, out_ref, idx_v, buf_v, gsem, osem):
    wid = lax.axis_index("s") * NC + lax.axis_index("c")
    ibase = pl.multiple_of(wid * IDX_W, IDX_W)

    pltpu.sync_copy(x_ref.at[pl.ds(ibase, IDX_W)], idx_v)

    def do_block(blk, _):
        buf = buf_v.at[lax.rem(blk, 2)]
        @pl.when(blk >= 2)
        def _wait_store():
            pltpu.make_async_copy(
                out_ref.at[pl.ds(0, BLK_IDX), :], buf, osem
            ).wait()

        p0 = pl.multiple_of(blk * BLK_IDX, BLK_IDX)
        for j in range(BLK_IDX // 16):
            vec = idx_v[pl.ds(p0 + j * 16, 16)]
            for l in range(16):
                i = (j * 16 + l) % N_FIELDS
                row = vec[l]
                pltpu.async_copy(
                    table_ref.at[i, pl.ds(row, 1), :],
                    buf.at[pl.ds(j * 16 + l, 1), :],
                    gsem,
                )
        pltpu.make_async_copy(out_ref.at[pl.ds(0, BLK_IDX), :], buf, gsem).wait()
        o = pl.multiple_of(ibase + p0, 8)
        pltpu.make_async_copy(buf, out_ref.at[pl.ds(o, BLK_IDX), :], osem).start()
        return _

    lax.fori_loop(0, NBLK, do_block, None)
    pltpu.make_async_copy(
        out_ref.at[pl.ds(0, BLK_IDX), :], buf_v.at[0], osem
    ).wait()
    pltpu.make_async_copy(
        out_ref.at[pl.ds(0, BLK_IDX), :], buf_v.at[1], osem
    ).wait()


def kernel(x_cat, tables):
    x_flat = x_cat.reshape(BATCH * N_FIELDS)
    out = _embed_gather(x_flat, tables)
    return out.reshape(BATCH, OUT_W)


# speedup vs baseline: 1.3922x; 1.3922x over previous
import functools

import jax
import jax.numpy as jnp
from jax import lax
from jax.experimental import pallas as pl
from jax.experimental.pallas import tpu as pltpu
from jax.experimental.pallas import tpu_sc as plsc

N_FIELDS = 26
CARD = 100000
DIM = 24
BATCH = 16384
OUT_W = N_FIELDS * DIM

NC = 2
NS = 16
NW = NC * NS
IDX_W = BATCH * N_FIELDS // NW
BLK = 16
NBLK = IDX_W // (BLK * N_FIELDS)
BLK_IDX = BLK * N_FIELDS

_mesh = plsc.VectorSubcoreMesh(core_axis_name="c", subcore_axis_name="s")


@functools.partial(
    pl.kernel,
    mesh=_mesh,
    out_type=jax.ShapeDtypeStruct((BATCH * N_FIELDS, DIM), jnp.float32),
    scratch_types=[
        pltpu.VMEM((IDX_W,), jnp.int32),
        pltpu.VMEM((2, BLK_IDX, DIM), jnp.float32),
        pltpu.SemaphoreType.DMA,
        pltpu.SemaphoreType.DMA,
    ],
)
def _embed_gather(x_ref, table_ref, out_ref, idx_v, buf_v, gsem, osem):
    wid = lax.axis_index("s") * NC + lax.axis_index("c")
    ibase = pl.multiple_of(wid * IDX_W, IDX_W)

    pltpu.sync_copy(x_ref.at[pl.ds(ibase, IDX_W)], idx_v)

    def add_offsets(g, _):
        b = pl.multiple_of(g * 208, 208)
        for j in range(13):
            pos = j * 16 + lax.iota(jnp.int32, 16)
            off = lax.rem(pos, N_FIELDS) * CARD
            idx_v[pl.ds(b + j * 16, 16)] = idx_v[pl.ds(b + j * 16, 16)] + off
        return _

    lax.fori_loop(0, IDX_W // 208, add_offsets, None)

    def do_block(blk, _):
        buf = buf_v.at[lax.rem(blk, 2)]
        @pl.when(blk >= 2)
        def _wait_store():
            pltpu.make_async_copy(
                out_ref.at[pl.ds(0, BLK_IDX), :], buf, osem
            ).wait()

        iblk = pl.multiple_of(blk * BLK_IDX, BLK_IDX)

        def fire_grp(g, _):
            vec = idx_v[pl.ds(pl.multiple_of(iblk + g * 16, 16), 16)]
            d = g * 16
            for l in range(16):
                row = vec[l]
                pltpu.async_copy(
                    table_ref.at[pl.ds(row, 1), :],
                    buf.at[pl.ds(d + l, 1), :],
                    gsem,
                )
            return _

        lax.fori_loop(0, BLK_IDX // 16, fire_grp, None)
        pltpu.make_async_copy(out_ref.at[pl.ds(0, BLK_IDX), :], buf, gsem).wait()
        o = pl.multiple_of(ibase + iblk, 8)
        pltpu.make_async_copy(buf, out_ref.at[pl.ds(o, BLK_IDX), :], osem).start()
        return _

    lax.fori_loop(0, NBLK, do_block, None)
    pltpu.make_async_copy(
        out_ref.at[pl.ds(0, BLK_IDX), :], buf_v.at[0], osem
    ).wait()
    pltpu.make_async_copy(
        out_ref.at[pl.ds(0, BLK_IDX), :], buf_v.at[1], osem
    ).wait()


def kernel(x_cat, tables):
    x_flat = x_cat.reshape(BATCH * N_FIELDS)
    table2d = tables.reshape(N_FIELDS * CARD, DIM)
    out = _embed_gather(x_flat, table2d)
    return out.reshape(BATCH, OUT_W)


# speedup vs baseline: 1.3982x; 1.0043x over previous
import functools

import jax
import jax.numpy as jnp
from jax import lax
from jax.experimental import pallas as pl
from jax.experimental.pallas import tpu as pltpu
from jax.experimental.pallas import tpu_sc as plsc

N_FIELDS = 26
CARD = 100000
DIM = 24
BATCH = 16384
OUT_W = N_FIELDS * DIM

NC = 2
NS = 16
NW = NC * NS
IDX_W = BATCH * N_FIELDS // NW
BLK = 16
NBLK = IDX_W // (BLK * N_FIELDS)
BLK_IDX = BLK * N_FIELDS

_mesh = plsc.VectorSubcoreMesh(core_axis_name="c", subcore_axis_name="s")


@functools.partial(
    pl.kernel,
    mesh=_mesh,
    out_type=jax.ShapeDtypeStruct((BATCH * N_FIELDS, DIM), jnp.float32),
    scratch_types=[
        pltpu.VMEM((IDX_W,), jnp.int32),
        pltpu.VMEM((2, BLK_IDX, DIM), jnp.float32),
        pltpu.SemaphoreType.DMA,
        pltpu.SemaphoreType.DMA,
        pltpu.SemaphoreType.DMA,
    ],
)
def _embed_gather(x_ref, table_ref, out_ref, idx_v, buf_v, gsem0, gsem1, osem):
    wid = lax.axis_index("s") * NC + lax.axis_index("c")
    ibase = pl.multiple_of(wid * IDX_W, IDX_W)

    pltpu.sync_copy(x_ref.at[pl.ds(ibase, IDX_W)], idx_v)

    def add_offsets(g, _):
        b = pl.multiple_of(g * 208, 208)
        for j in range(13):
            pos = j * 16 + lax.iota(jnp.int32, 16)
            off = lax.rem(pos, N_FIELDS) * CARD
            idx_v[pl.ds(b + j * 16, 16)] = idx_v[pl.ds(b + j * 16, 16)] + off
        return _

    lax.fori_loop(0, IDX_W // 208, add_offsets, None)

    def fire(blk, buf, gsem):
        iblk = pl.multiple_of(blk * BLK_IDX, BLK_IDX)

        def fire_grp(g, _):
            vec = idx_v[pl.ds(pl.multiple_of(iblk + g * 16, 16), 16)]
            d = g * 16
            for l in range(16):
                row = vec[l]
                pltpu.async_copy(
                    table_ref.at[pl.ds(row, 1), :],
                    buf.at[pl.ds(d + l, 1), :],
                    gsem,
                )
            return _

        lax.fori_loop(0, BLK_IDX // 16, fire_grp, None)

    def drain_store(blk, buf, gsem):
        pltpu.make_async_copy(out_ref.at[pl.ds(0, BLK_IDX), :], buf, gsem).wait()
        o = pl.multiple_of(ibase + blk * BLK_IDX, 8)
        pltpu.make_async_copy(buf, out_ref.at[pl.ds(o, BLK_IDX), :], osem).start()

    def wait_store(buf):
        pltpu.make_async_copy(out_ref.at[pl.ds(0, BLK_IDX), :], buf, osem).wait()

    fire(0, buf_v.at[0], gsem0)
    fire(1, buf_v.at[1], gsem1)
    drain_store(0, buf_v.at[0], gsem0)

    def do_super(s, _):
        a = 2 * s
        wait_store(buf_v.at[0])
        fire(a, buf_v.at[0], gsem0)
        drain_store(a - 1, buf_v.at[1], gsem1)
        wait_store(buf_v.at[1])
        fire(a + 1, buf_v.at[1], gsem1)
        drain_store(a, buf_v.at[0], gsem0)
        return _

    lax.fori_loop(1, NBLK // 2, do_super, None)
    drain_store(NBLK - 1, buf_v.at[1], gsem1)
    wait_store(buf_v.at[0])
    wait_store(buf_v.at[1])


def kernel(x_cat, tables):
    x_flat = x_cat.reshape(BATCH * N_FIELDS)
    table2d = tables.reshape(N_FIELDS * CARD, DIM)
    out = _embed_gather(x_flat, table2d)
    return out.reshape(BATCH, OUT_W)
